# baseline (speedup 1.0000x reference)
import functools

import jax
import jax.numpy as jnp
from jax import lax
from jax.experimental import pallas as pl
from jax.experimental.pallas import tpu as pltpu
from jax.experimental.pallas import tpu_sc as plsc

_VOCAB = 100000
_HIDDEN = 128
_MASK_ID = 103
_K = 5
_NSLOT = 8

_TB = 64
_VC = 4096
_NEG = -1.0
_BIGI = jnp.iinfo(jnp.int32).max


def _sigmoid(x):
    e = jnp.exp(-jnp.abs(x))
    return jnp.where(x >= 0, 1.0 / (1.0 + e), e / (1.0 + e))


def _tc_body(xt_ref, p_ref, sa_ref, sb_ref, ss_ref, gi_ref, gw_ref,
             rv, ri, ea):
    v = pl.program_id(1)
    nv = pl.num_programs(1)

    @pl.when(v == 0)
    def _init():
        rv[...] = jnp.full((_TB, _NSLOT), _NEG, jnp.float32)
        ri[...] = jnp.full((_TB, _NSLOT), _BIGI, jnp.int32)
        ea[...] = jnp.zeros((_TB, 1), jnp.float32)

    p = p_ref[...]
    gpos = v * _VC + lax.broadcasted_iota(jnp.int32, (_TB, _VC), 1)
    valid = gpos < _VOCAB
    pz = jnp.where(valid, p, 0.0)

    ent = jnp.where(pz > 0, -pz * jnp.log(jnp.where(pz > 0, pz, 1.0)), 0.0)
    ea[...] += jnp.sum(ent, axis=1, keepdims=True)

    cat_v = jnp.concatenate([jnp.where(valid, p, _NEG), rv[...]], axis=1)
    cat_i = jnp.concatenate([gpos, ri[...]], axis=1)
    vs, idxs = [], []
    for _ in range(_K):
        m = jnp.max(cat_v, axis=1, keepdims=True)
        pos = jnp.min(jnp.where(cat_v == m, cat_i, _BIGI), axis=1,
                      keepdims=True)
        vs.append(m)
        idxs.append(pos)
        cat_v = jnp.where(cat_i == pos, _NEG, cat_v)
    pad_v = jnp.full((_TB, 1), _NEG, jnp.float32)
    pad_i = jnp.full((_TB, 1), _BIGI, jnp.int32)
    rv[...] = jnp.concatenate(vs + [pad_v] * (_NSLOT - _K), axis=1)
    ri[...] = jnp.concatenate(idxs + [pad_i] * (_NSLOT - _K), axis=1)

    @pl.when(v == nv - 1)
    def _finalize():
        li = lax.broadcasted_iota(jnp.int32, (_TB, _NSLOT), 1)
        tv = rv[...]
        ti = ri[...]
        s = jnp.sum(jnp.where(li < _K, tv, 0.0), axis=1, keepdims=True)
        pn = tv / (s + 1e-10)
        a = sa_ref[0, 0]
        b = sb_ref[0, 0]
        sg = ss_ref[0, 0]
        lam = sg * _sigmoid(a * (-ea[...] - b))
        xt = xt_ref[...]
        ism = xt == _MASK_ID
        w_mask = jnp.where(li < _K, lam * pn,
                           jnp.where(li == _K, 1.0 - lam, 0.0))
        w_real = jnp.where(li == 0, 1.0, 0.0)
        gw_ref[...] = jnp.where(ism, w_mask, w_real)
        i_mask = jnp.where(li < _K, ti,
                           jnp.where(li == _K, _MASK_ID, 0))
        i_real = jnp.where(li == 0, xt, 0)
        gi_ref[...] = jnp.where(ism, i_mask, i_real)


def _tc_stage(xt2, p2, sa, sb, sg):
    n = xt2.shape[0]
    nt = n // _TB
    nv = pl.cdiv(_VOCAB, _VC)
    grid = (nt, nv)
    return pl.pallas_call(
        _tc_body,
        grid=grid,
        in_specs=[
            pl.BlockSpec((_TB, 1), lambda t, v: (t, 0)),
            pl.BlockSpec((_TB, _VC), lambda t, v: (t, v)),
            pl.BlockSpec(memory_space=pltpu.SMEM),
            pl.BlockSpec(memory_space=pltpu.SMEM),
            pl.BlockSpec(memory_space=pltpu.SMEM),
        ],
        out_specs=[
            pl.BlockSpec((_TB, _NSLOT), lambda t, v: (t, 0)),
            pl.BlockSpec((_TB, _NSLOT), lambda t, v: (t, 0)),
        ],
        out_shape=[
            jax.ShapeDtypeStruct((n, _NSLOT), jnp.int32),
            jax.ShapeDtypeStruct((n, _NSLOT), jnp.float32),
        ],
        scratch_shapes=[
            pltpu.VMEM((_TB, _NSLOT), jnp.float32),
            pltpu.VMEM((_TB, _NSLOT), jnp.int32),
            pltpu.VMEM((_TB, 1), jnp.float32),
        ],
        compiler_params=pltpu.CompilerParams(
            dimension_semantics=("parallel", "arbitrary")),
    )(xt2, p2, sa, sb, sg)


def _sc_combine(table, gi_flat, w_r, n):
    info = plsc.get_sparse_core_info()
    nc, ns, nl = info.num_cores, info.num_subcores, info.num_lanes
    nw = nc * ns
    tpw = n // nw
    mesh = plsc.VectorSubcoreMesh(core_axis_name="c", subcore_axis_name="s")

    @functools.partial(
        pl.kernel,
        mesh=mesh,
        out_type=jax.ShapeDtypeStruct((n, _HIDDEN), jnp.float32),
        scratch_types=[
            pltpu.VMEM((_NSLOT * tpw,), jnp.int32),
            pltpu.VMEM((_NSLOT, tpw, _HIDDEN), jnp.float32),
            pltpu.VMEM((_NSLOT, tpw, _HIDDEN), jnp.float32),
            pltpu.VMEM((tpw, _HIDDEN), jnp.float32),
            pltpu.SemaphoreType.DMA,
        ],
    )
    def k(tab_hbm, gi_hbm, w_hbm, out_hbm, idx_v, w_v, rows_v, out_v, sem):
        wid = lax.axis_index("s") * nc + lax.axis_index("c")
        base = wid * tpw
        pltpu.sync_copy(gi_hbm.at[pl.ds(wid * _NSLOT * tpw, _NSLOT * tpw)],
                        idx_v)
        pltpu.sync_copy(w_hbm.at[wid], w_v)
        for s in range(_NSLOT):
            pltpu.async_copy(tab_hbm.at[idx_v.at[pl.ds(s * tpw, tpw)]],
                             rows_v.at[s], sem).wait()

        def body(j, carry):
            for c in range(_HIDDEN // nl):
                sl = pl.ds(c * nl, nl)
                acc = w_v[0, j, sl] * rows_v[0, j, sl]
                for s in range(1, _NSLOT):
                    acc = acc + w_v[s, j, sl] * rows_v[s, j, sl]
                out_v[j, sl] = acc
            return carry

        lax.fori_loop(0, tpw, body, 0)
        pltpu.sync_copy(out_v, out_hbm.at[pl.ds(base, tpw)])

    return k(table, gi_flat, w_r)


def kernel(x_t, probs, embedding_weight, omega_s, omega_a, omega_b):
    bsz, seq = x_t.shape
    n = bsz * seq
    p2 = probs.reshape(n, _VOCAB)
    xt2 = x_t.reshape(n, 1).astype(jnp.int32)
    sg = jax.nn.sigmoid(omega_s).astype(jnp.float32).reshape(1, 1)
    sa = jax.nn.softplus(omega_a).astype(jnp.float32).reshape(1, 1)
    sb = (-jax.nn.softplus(omega_b)).astype(jnp.float32).reshape(1, 1)
    gi, gw = _tc_stage(xt2, p2, sa, sb, sg)
    if True:
        rows = jnp.take(embedding_weight, gi, axis=0)
        out = jnp.sum(rows * gw[..., None], axis=1)
        return out.reshape(bsz, seq, _HIDDEN)
    info = plsc.get_sparse_core_info()
    nw = info.num_cores * info.num_subcores
    tpw = n // nw
    gi_flat = gi.reshape(nw, tpw, _NSLOT).transpose(0, 2, 1).reshape(-1)
    gw_r = gw.reshape(nw, tpw, _NSLOT).transpose(0, 2, 1)
    w_r = jnp.broadcast_to(gw_r[..., None], (nw, _NSLOT, tpw, _HIDDEN))
    out = _sc_combine(embedding_weight, gi_flat, w_r, n)
    return out.reshape(bsz, seq, _HIDDEN)


# speedup vs baseline: 1.0391x; 1.0391x over previous
import functools

import jax
import jax.numpy as jnp
from jax import lax
from jax.experimental import pallas as pl
from jax.experimental.pallas import tpu as pltpu
from jax.experimental.pallas import tpu_sc as plsc

_VOCAB = 100000
_HIDDEN = 128
_MASK_ID = 103
_K = 5
_NSLOT = 8

_TB = 64
_VC = 4096
_NEG = -1.0
_BIGI = jnp.iinfo(jnp.int32).max


def _sigmoid(x):
    e = jnp.exp(-jnp.abs(x))
    return jnp.where(x >= 0, 1.0 / (1.0 + e), e / (1.0 + e))


_BIGF = 1.0e9


def _tc_body(xt_ref, p_ref, sa_ref, sb_ref, ss_ref, gi_ref, gw_ref,
             rv, ri, ea):
    v = pl.program_id(1)
    nv = pl.num_programs(1)

    @pl.when(v == 0)
    def _init():
        rv[...] = jnp.full((_TB, _NSLOT), _NEG, jnp.float32)
        ri[...] = jnp.full((_TB, _NSLOT), _BIGF, jnp.float32)
        ea[...] = jnp.zeros((_TB, 1), jnp.float32)

    p = p_ref[...]
    gposf = (jnp.float32(v * _VC)
             + lax.broadcasted_iota(jnp.int32, (_TB, _VC), 1)
             .astype(jnp.float32))
    w0 = jnp.where(gposf < jnp.float32(_VOCAB), p, _NEG)

    pz = jnp.maximum(w0, 0.0)
    lg = jnp.log(jnp.maximum(w0, 1e-30))
    ea[...] -= jnp.sum(pz * lg, axis=1, keepdims=True)

    wk = w0
    vs, idxs = [], []
    for _ in range(_K):
        m = jnp.max(wk, axis=1, keepdims=True)
        pos = jnp.min(jnp.where(wk == m, gposf, _BIGF), axis=1,
                      keepdims=True)
        vs.append(m)
        idxs.append(pos)
        wk = jnp.where(gposf == pos, _NEG, wk)

    pad_v = jnp.full((_TB, 1), _NEG, jnp.float32)
    pad_i = jnp.full((_TB, 1), _BIGF, jnp.float32)
    cat_v = jnp.concatenate([rv[...]] + vs + [pad_v] * (_NSLOT - _K), axis=1)
    cat_i = jnp.concatenate([ri[...]] + idxs + [pad_i] * (_NSLOT - _K),
                            axis=1)
    mvs, mis = [], []
    for _ in range(_K):
        m = jnp.max(cat_v, axis=1, keepdims=True)
        pos = jnp.min(jnp.where(cat_v == m, cat_i, _BIGF), axis=1,
                      keepdims=True)
        mvs.append(m)
        mis.append(pos)
        cat_v = jnp.where(cat_i == pos, _NEG, cat_v)
    rv[...] = jnp.concatenate(mvs + [pad_v] * (_NSLOT - _K), axis=1)
    ri[...] = jnp.concatenate(mis + [pad_i] * (_NSLOT - _K), axis=1)

    @pl.when(v == nv - 1)
    def _finalize():
        li = lax.broadcasted_iota(jnp.int32, (_TB, _NSLOT), 1)
        tv = rv[...]
        ti = ri[...].astype(jnp.int32)
        s = jnp.sum(jnp.where(li < _K, tv, 0.0), axis=1, keepdims=True)
        pn = tv / (s + 1e-10)
        a = sa_ref[0, 0]
        b = sb_ref[0, 0]
        sg = ss_ref[0, 0]
        lam = sg * _sigmoid(a * (-ea[...] - b))
        xt = xt_ref[...]
        ism = xt == _MASK_ID
        w_mask = jnp.where(li < _K, lam * pn,
                           jnp.where(li == _K, 1.0 - lam, 0.0))
        w_real = jnp.where(li == 0, 1.0, 0.0)
        gw_ref[...] = jnp.where(ism, w_mask, w_real)
        i_mask = jnp.where(li < _K, ti,
                           jnp.where(li == _K, _MASK_ID, 0))
        i_real = jnp.where(li == 0, xt, 0)
        gi_ref[...] = jnp.where(ism, i_mask, i_real)


def _tc_stage(xt2, p2, sa, sb, sg):
    n = xt2.shape[0]
    nt = n // _TB
    nv = pl.cdiv(_VOCAB, _VC)
    grid = (nt, nv)
    return pl.pallas_call(
        _tc_body,
        grid=grid,
        in_specs=[
            pl.BlockSpec((_TB, 1), lambda t, v: (t, 0)),
            pl.BlockSpec((_TB, _VC), lambda t, v: (t, v)),
            pl.BlockSpec(memory_space=pltpu.SMEM),
            pl.BlockSpec(memory_space=pltpu.SMEM),
            pl.BlockSpec(memory_space=pltpu.SMEM),
        ],
        out_specs=[
            pl.BlockSpec((_TB, _NSLOT), lambda t, v: (t, 0)),
            pl.BlockSpec((_TB, _NSLOT), lambda t, v: (t, 0)),
        ],
        out_shape=[
            jax.ShapeDtypeStruct((n, _NSLOT), jnp.int32),
            jax.ShapeDtypeStruct((n, _NSLOT), jnp.float32),
        ],
        scratch_shapes=[
            pltpu.VMEM((_TB, _NSLOT), jnp.float32),
            pltpu.VMEM((_TB, _NSLOT), jnp.float32),
            pltpu.VMEM((_TB, 1), jnp.float32),
        ],
        compiler_params=pltpu.CompilerParams(
            dimension_semantics=("parallel", "arbitrary")),
    )(xt2, p2, sa, sb, sg)


def _sc_combine(table, gi_flat, w_r, n):
    info = plsc.get_sparse_core_info()
    nc, ns, nl = info.num_cores, info.num_subcores, info.num_lanes
    nw = nc * ns
    tpw = n // nw
    mesh = plsc.VectorSubcoreMesh(core_axis_name="c", subcore_axis_name="s")

    @functools.partial(
        pl.kernel,
        mesh=mesh,
        out_type=jax.ShapeDtypeStruct((n, _HIDDEN), jnp.float32),
        scratch_types=[
            pltpu.VMEM((_NSLOT * tpw,), jnp.int32),
            pltpu.VMEM((_NSLOT, tpw, _HIDDEN), jnp.float32),
            pltpu.VMEM((_NSLOT, tpw, _HIDDEN), jnp.float32),
            pltpu.VMEM((tpw, _HIDDEN), jnp.float32),
            pltpu.SemaphoreType.DMA,
        ],
    )
    def k(tab_hbm, gi_hbm, w_hbm, out_hbm, idx_v, w_v, rows_v, out_v, sem):
        wid = lax.axis_index("s") * nc + lax.axis_index("c")
        base = wid * tpw
        pltpu.sync_copy(gi_hbm.at[pl.ds(wid * _NSLOT * tpw, _NSLOT * tpw)],
                        idx_v)
        pltpu.sync_copy(w_hbm.at[wid], w_v)
        for s in range(_NSLOT):
            pltpu.async_copy(tab_hbm.at[idx_v.at[pl.ds(s * tpw, tpw)]],
                             rows_v.at[s], sem).wait()

        def body(j, carry):
            for c in range(_HIDDEN // nl):
                sl = pl.ds(c * nl, nl)
                acc = w_v[0, j, sl] * rows_v[0, j, sl]
                for s in range(1, _NSLOT):
                    acc = acc + w_v[s, j, sl] * rows_v[s, j, sl]
                out_v[j, sl] = acc
            return carry

        lax.fori_loop(0, tpw, body, 0)
        pltpu.sync_copy(out_v, out_hbm.at[pl.ds(base, tpw)])

    return k(table, gi_flat, w_r)


def kernel(x_t, probs, embedding_weight, omega_s, omega_a, omega_b):
    bsz, seq = x_t.shape
    n = bsz * seq
    p2 = probs.reshape(n, _VOCAB)
    xt2 = x_t.reshape(n, 1).astype(jnp.int32)
    sg = jax.nn.sigmoid(omega_s).astype(jnp.float32).reshape(1, 1)
    sa = jax.nn.softplus(omega_a).astype(jnp.float32).reshape(1, 1)
    sb = (-jax.nn.softplus(omega_b)).astype(jnp.float32).reshape(1, 1)
    gi, gw = _tc_stage(xt2, p2, sa, sb, sg)
    info = plsc.get_sparse_core_info()
    nw = info.num_cores * info.num_subcores
    tpw = n // nw
    gi_flat = gi.reshape(nw, tpw, _NSLOT).transpose(0, 2, 1).reshape(-1)
    gw_r = gw.reshape(nw, tpw, _NSLOT).transpose(0, 2, 1)
    w_r = jnp.broadcast_to(gw_r[..., None], (nw, _NSLOT, tpw, _HIDDEN))
    out = _sc_combine(embedding_weight, gi_flat, w_r, n)
    return out.reshape(bsz, seq, _HIDDEN)
